# baseline (speedup 1.0000x reference)
import functools

import jax
import jax.numpy as jnp
import numpy as np
from jax import lax
from jax.experimental import pallas as pl
from jax.experimental.pallas import tpu as pltpu
from jax.experimental.pallas import tpu_sc as plsc

_B, _S, _K, _D = 1024, 16, 64, 32
_ROWS = _B * _S
_R = 512
_NW = 32
_RPW = _ROWS // _NW
_IDXW = 128
_JW = _RPW // _IDXW


def _sample_body(bits_ref, lp_ref, out_ref):
    bits = bits_ref[...]
    fb = (bits >> jnp.uint32(9)) | jnp.uint32(0x3F800000)
    f = lax.bitcast_convert_type(fb, jnp.float32) - jnp.float32(1.0)
    tiny = jnp.float32(np.finfo(np.float32).tiny)
    u = jnp.maximum(tiny, f * (jnp.float32(1.0) - tiny) + tiny)
    g = -jnp.log(-jnp.log(u))
    s = g + lp_ref[...]
    m = jnp.max(s, axis=1, keepdims=True)
    ik = lax.broadcasted_iota(jnp.int32, (_R, _K), 1)
    z = jnp.min(jnp.where(s == m, ik, jnp.int32(_K)), axis=1, keepdims=True)
    row = pl.program_id(0) * _R + lax.broadcasted_iota(jnp.int32, (_R, 1), 0)
    out_ref[...] = row * _K + z


def _sc_gather(mus_hbm, sig_hbm, idx_hbm, omu_hbm, osg_hbm,
               idx_v, mu_v, sg_v, sem):
    wid = lax.axis_index("s") * 2 + lax.axis_index("c")
    pltpu.sync_copy(idx_hbm.at[pl.ds(wid * _JW, _JW)], idx_v)
    cps = []
    for j in range(_JW):
        dst = pl.ds(j * _IDXW, _IDXW)
        cps.append(pltpu.async_copy(mus_hbm.at[idx_v.at[j]], mu_v.at[dst], sem))
        cps.append(pltpu.async_copy(sig_hbm.at[idx_v.at[j]], sg_v.at[dst], sem))
    for cp in cps:
        cp.wait()
    out = pl.ds(wid * _RPW, _RPW)
    pltpu.sync_copy(mu_v, omu_hbm.at[out])
    pltpu.sync_copy(sg_v, osg_hbm.at[out])


def _gather_call():
    return functools.partial(
        pl.kernel,
        out_type=[jax.ShapeDtypeStruct((_ROWS, _D), jnp.float32),
                  jax.ShapeDtypeStruct((_ROWS, _D), jnp.float32)],
        mesh=plsc.VectorSubcoreMesh(core_axis_name="c", subcore_axis_name="s"),
        scratch_types=[pltpu.VMEM((_JW, _IDXW), jnp.int32),
                       pltpu.VMEM((_RPW, _D), jnp.float32),
                       pltpu.VMEM((_RPW, _D), jnp.float32),
                       pltpu.SemaphoreType.DMA],
        compiler_params=pltpu.CompilerParams(use_tc_tiling_on_sc=False),
    )


def kernel(mus, sigmas, pi):
    zkey = jax.random.fold_in(jax.random.key(0), 123)
    bits = jax.random.bits(zkey, (_B, _S, _K), jnp.uint32).reshape(_ROWS, _K)
    lp_block = jnp.tile(jnp.log(pi), (_R // _S, 1))

    flat_idx = pl.pallas_call(
        _sample_body,
        grid=(_ROWS // _R,),
        in_specs=[
            pl.BlockSpec((_R, _K), lambda i: (i, 0)),
            pl.BlockSpec((_R, _K), lambda i: (0, 0)),
        ],
        out_specs=pl.BlockSpec((_R, 1), lambda i: (i, 0)),
        out_shape=jax.ShapeDtypeStruct((_ROWS, 1), jnp.int32),
    )(bits, lp_block)

    idx_tiles = flat_idx.reshape(_NW * _JW, _IDXW)
    omu, osg = _gather_call()(_sc_gather)(
        mus.reshape(_ROWS * _K, _D), sigmas.reshape(_ROWS * _K, _D), idx_tiles)
    return omu.reshape(_B, _S, _D), osg.reshape(_B, _S, _D)


# speedup vs baseline: 4.7302x; 4.7302x over previous
import jax
import jax.numpy as jnp
import numpy as np
from jax import lax
from jax.experimental import pallas as pl
from jax.experimental.pallas import tpu as pltpu
from jax.experimental.pallas import tpu_sc as plsc

_B, _S, _K, _D = 1024, 16, 64, 32
_ROWS = _B * _S
_R = 512
_BB = 128


def _sample_body(bits_ref, lp_ref, out_ref):
    bits = bits_ref[...]
    fb = (bits >> jnp.uint32(9)) | jnp.uint32(0x3F800000)
    f = lax.bitcast_convert_type(fb, jnp.float32) - jnp.float32(1.0)
    tiny = jnp.float32(np.finfo(np.float32).tiny)
    u = jnp.maximum(tiny, f * (jnp.float32(1.0) - tiny) + tiny)
    g = -jnp.log(-jnp.log(u))
    s = g + lp_ref[...]
    m = jnp.max(s, axis=1, keepdims=True)
    ik = lax.broadcasted_iota(jnp.int32, (_R, _K), 1)
    out_ref[...] = jnp.min(jnp.where(s == m, ik, jnp.int32(_K)),
                           axis=1, keepdims=True)


def _select_body(mu_ref, sg_ref, z_ref, omu_ref, osg_ref):
    zrow = z_ref[0]
    acc_mu = mu_ref[0, 0]
    acc_sg = sg_ref[0, 0]
    for k in range(1, _K):
        mask = zrow == jnp.int32(k)
        acc_mu = jnp.where(mask, mu_ref[0, k], acc_mu)
        acc_sg = jnp.where(mask, sg_ref[0, k], acc_sg)
    omu_ref[0] = acc_mu
    osg_ref[0] = acc_sg


def kernel(mus, sigmas, pi):
    zkey = jax.random.fold_in(jax.random.key(0), 123)
    bits = jax.random.bits(zkey, (_B, _S, _K), jnp.uint32).reshape(_ROWS, _K)
    lp_block = jnp.tile(jnp.log(pi), (_R // _S, 1))

    z_flat = pl.pallas_call(
        _sample_body,
        grid=(_ROWS // _R,),
        in_specs=[
            pl.BlockSpec((_R, _K), lambda i: (i, 0)),
            pl.BlockSpec((_R, _K), lambda i: (0, 0)),
        ],
        out_specs=pl.BlockSpec((_R, 1), lambda i: (i, 0)),
        out_shape=jax.ShapeDtypeStruct((_ROWS, 1), jnp.int32),
    )(bits, lp_block)
    z_sb = z_flat.reshape(_B, _S).T.reshape(_S, 1, _B)

    mus_t = mus.transpose(1, 2, 3, 0)
    sig_t = sigmas.transpose(1, 2, 3, 0)

    omu_t, osg_t = pl.pallas_call(
        _select_body,
        grid=(_S, _B // _BB),
        in_specs=[
            pl.BlockSpec((1, _K, _D, _BB), lambda s, b: (s, 0, 0, b)),
            pl.BlockSpec((1, _K, _D, _BB), lambda s, b: (s, 0, 0, b)),
            pl.BlockSpec((1, 1, _BB), lambda s, b: (s, 0, b)),
        ],
        out_specs=[
            pl.BlockSpec((1, _D, _BB), lambda s, b: (s, 0, b)),
            pl.BlockSpec((1, _D, _BB), lambda s, b: (s, 0, b)),
        ],
        out_shape=[jax.ShapeDtypeStruct((_S, _D, _B), jnp.float32),
                   jax.ShapeDtypeStruct((_S, _D, _B), jnp.float32)],
    )(mus_t, sig_t, z_sb)

    return omu_t.transpose(2, 0, 1), osg_t.transpose(2, 0, 1)


# speedup vs baseline: 5.6701x; 1.1987x over previous
import jax
import jax.numpy as jnp
import numpy as np
from jax import lax
from jax.experimental import pallas as pl
from jax.experimental.pallas import tpu as pltpu
from jax.experimental.pallas import tpu_sc as plsc

_B, _S, _K, _D = 1024, 16, 64, 32
_ROWS = _B * _S
_R = 512
_BB = 256


def _sample_body(bits_ref, lp_ref, out_ref):
    bits = bits_ref[...]
    fb = (bits >> jnp.uint32(9)) | jnp.uint32(0x3F800000)
    f = lax.bitcast_convert_type(fb, jnp.float32) - jnp.float32(1.0)
    tiny = jnp.float32(np.finfo(np.float32).tiny)
    u = jnp.maximum(tiny, f * (jnp.float32(1.0) - tiny) + tiny)
    g = -jnp.log(-jnp.log(u))
    s = g + lp_ref[...]
    m = jnp.max(s, axis=1, keepdims=True)
    ik = lax.broadcasted_iota(jnp.int32, (_R, _K), 1)
    out_ref[...] = jnp.min(jnp.where(s == m, ik, jnp.int32(_K)),
                           axis=1, keepdims=True)


def _select_body(mu_ref, sg_ref, z_ref, omu_ref, osg_ref):
    zrow = z_ref[0]
    acc_mu = mu_ref[0, 0]
    acc_sg = sg_ref[0, 0]
    for k in range(1, _K):
        mask = zrow == jnp.int32(k)
        acc_mu = jnp.where(mask, mu_ref[0, k], acc_mu)
        acc_sg = jnp.where(mask, sg_ref[0, k], acc_sg)
    omu_ref[0] = acc_mu
    osg_ref[0] = acc_sg


def kernel(mus, sigmas, pi):
    zkey = jax.random.fold_in(jax.random.key(0), 123)
    bits = jax.random.bits(zkey, (_B, _S, _K), jnp.uint32).reshape(_ROWS, _K)
    lp_block = jnp.tile(jnp.log(pi), (_R // _S, 1))

    z_flat = pl.pallas_call(
        _sample_body,
        grid=(_ROWS // _R,),
        in_specs=[
            pl.BlockSpec((_R, _K), lambda i: (i, 0)),
            pl.BlockSpec((_R, _K), lambda i: (0, 0)),
        ],
        out_specs=pl.BlockSpec((_R, 1), lambda i: (i, 0)),
        out_shape=jax.ShapeDtypeStruct((_ROWS, 1), jnp.int32),
    )(bits, lp_block)
    z_sb = z_flat.reshape(_B, _S).T.reshape(_S, 1, _B)

    mus_t = mus.transpose(1, 2, 3, 0)
    sig_t = sigmas.transpose(1, 2, 3, 0)

    omu_t, osg_t = pl.pallas_call(
        _select_body,
        grid=(_S, _B // _BB),
        in_specs=[
            pl.BlockSpec((1, _K, _D, _BB), lambda s, b: (s, 0, 0, b)),
            pl.BlockSpec((1, _K, _D, _BB), lambda s, b: (s, 0, 0, b)),
            pl.BlockSpec((1, 1, _BB), lambda s, b: (s, 0, b)),
        ],
        out_specs=[
            pl.BlockSpec((1, _D, _BB), lambda s, b: (s, 0, b)),
            pl.BlockSpec((1, _D, _BB), lambda s, b: (s, 0, b)),
        ],
        out_shape=[jax.ShapeDtypeStruct((_S, _D, _B), jnp.float32),
                   jax.ShapeDtypeStruct((_S, _D, _B), jnp.float32)],
    )(mus_t, sig_t, z_sb)

    return omu_t.transpose(2, 0, 1), osg_t.transpose(2, 0, 1)
